# baseline (speedup 1.0000x reference)
import jax
import jax.numpy as jnp
from jax.experimental import pallas as pl
from jax.experimental.pallas import tpu as pltpu

_NUM_CLASSES = 30
_LO = 1.0
_HI = 799.0
_BLK = 128
_SUB = 32
_NBUF = 12


def _body(pt_ref, gt_ref, ph_ref, lab_ref, x_ref, cn_ref, mt_ref, pf_ref,
          *rest):
    bufs = rest[:_NBUF]
    buft = rest[_NBUF]
    sems = rest[_NBUF + 1:2 * _NBUF + 1]
    semt = rest[2 * _NBUF + 1]
    n = pt_ref.shape[1]
    nfull = (n // _BLK) * _BLK

    px1 = jnp.clip(pt_ref[0:1, :], _LO, _HI)
    py1 = jnp.clip(pt_ref[1:2, :], _LO, _HI)
    px2 = jnp.clip(pt_ref[2:3, :], _LO, _HI)
    py2 = jnp.clip(pt_ref[3:4, :], _LO, _HI)
    area_b = (px2 - px1 + 1.0) * (py2 - py1 + 1.0)

    iou_rows = []
    for g in range(8):
        gx1 = jnp.clip(gt_ref[g, 0], _LO, _HI)
        gy1 = jnp.clip(gt_ref[g, 1], _LO, _HI)
        gx2 = jnp.clip(gt_ref[g, 2], _LO, _HI)
        gy2 = jnp.clip(gt_ref[g, 3], _LO, _HI)
        iw = jnp.maximum(jnp.minimum(px2, gx2) - jnp.maximum(px1, gx1)
                         + 1.0, 0.0)
        ih = jnp.maximum(jnp.minimum(py2, gy2) - jnp.maximum(py1, gy1)
                         + 1.0, 0.0)
        inter = iw * ih
        area_g = (gx2 - gx1 + 1.0) * (gy2 - gy1 + 1.0)
        iou_rows.append(inter / (area_b + area_g - inter))

    cls_iota = jax.lax.broadcasted_iota(jnp.int32, (32, 1), 0)
    cn = jnp.zeros((32, n), jnp.float32)
    for g in range(8):
        onehot = (cls_iota == lab_ref[g]).astype(jnp.float32)
        cn = jnp.maximum(cn, onehot * iou_rows[g])
    cn_ref[...] = jnp.transpose(cn)[:, :_NUM_CLASSES]

    mrows = []
    for g in range(8):
        acc = iou_rows[g]
        for g2 in range(8):
            if g2 == g:
                continue
            same = lab_ref[g] == lab_ref[g2]
            acc = jnp.maximum(acc, jnp.where(same, iou_rows[g2], 0.0))
        mrows.append((acc > 0.6).astype(jnp.float32))
    mask = jnp.concatenate(mrows, axis=0)

    src_w = px2 - px1
    src_h = py2 - py1
    src_cx = px1 + 0.5 * src_w
    src_cy = py1 + 0.5 * src_h
    rows = []
    for g in range(8):
        hx1 = jnp.clip(ph_ref[g, 0], _LO, _HI)
        hy1 = jnp.clip(ph_ref[g, 1], _LO, _HI)
        hx2 = jnp.clip(ph_ref[g, 2], _LO, _HI)
        hy2 = jnp.clip(ph_ref[g, 3], _LO, _HI)
        gw = hx2 - hx1
        gh = hy2 - hy1
        gcx = hx1 + 0.5 * gw
        gcy = hy1 + 0.5 * gh
        m = mrows[g]
        rows.append(((gcx - src_cx) / src_w) * m)
        rows.append(((gcy - src_cy) / src_h) * m)
        rows.append(jnp.log(gw / src_w) * m)
        rows.append(jnp.log(gh / src_h) * m)
    mt_ref[...] = jnp.concatenate(rows, axis=0)

    pf_ref[...] = jnp.zeros(pf_ref.shape, jnp.float32)

    nb = nfull // _BLK
    nsub = _BLK // _SUB
    subs = [[mask[:, b * _BLK + s * _SUB:b * _BLK + (s + 1) * _SUB]
             for s in range(nsub)] for b in range(nb)]
    sflags = [[jnp.max(subs[b][s]) > 0.5 for s in range(nsub)]
              for b in range(nb)]

    def start(b):
        for s in range(nsub):
            @pl.when(sflags[b][s])
            def _(s=s):
                pltpu.make_async_copy(
                    x_ref.at[pl.ds(b * _BLK + s * _SUB, _SUB), :],
                    bufs[b % _NBUF].at[pl.ds(s * _SUB, _SUB), :],
                    sems[b % _NBUF]).start()

    def finish(b):
        for s in range(nsub):
            @pl.when(sflags[b][s])
            def _(s=s):
                pltpu.make_async_copy(
                    x_ref.at[pl.ds(b * _BLK + s * _SUB, _SUB), :],
                    bufs[b % _NBUF].at[pl.ds(s * _SUB, _SUB), :],
                    sems[b % _NBUF]).wait()
                pf_ref[...] += jnp.dot(
                    subs[b][s], bufs[b % _NBUF][pl.ds(s * _SUB, _SUB), :],
                    preferred_element_type=jnp.float32)

    pending = []
    for b in range(nb):
        start(b)
        pending.append(b)
        if len(pending) == _NBUF:
            finish(pending.pop(0))
    for b in pending:
        finish(b)

    if nfull < n:
        tail = mask[:, nfull:n]
        tflag = jnp.max(tail) > 0.5

        @pl.when(tflag)
        def _():
            cp = pltpu.make_async_copy(
                x_ref.at[pl.ds(nfull, n - nfull), :], buft, semt)
            cp.start()
            cp.wait()
            pf_ref[...] += jnp.dot(tail, buft[...],
                                   preferred_element_type=jnp.float32)


def kernel(x, proposals, gt_bbox, gt_labels):
    n, d = x.shape
    g = gt_bbox.shape[0]
    labs = gt_labels.astype(jnp.int32)
    pt = proposals.T
    ph = proposals[:g]

    cn, mt, pf = pl.pallas_call(
        _body,
        grid=(1,),
        in_specs=[
            pl.BlockSpec((4, n), lambda i: (0, 0)),
            pl.BlockSpec(memory_space=pltpu.SMEM),
            pl.BlockSpec(memory_space=pltpu.SMEM),
            pl.BlockSpec(memory_space=pltpu.SMEM),
            pl.BlockSpec(memory_space=pltpu.MemorySpace.HBM),
        ],
        out_specs=[
            pl.BlockSpec((n, _NUM_CLASSES), lambda i: (0, 0)),
            pl.BlockSpec((32, n), lambda i: (0, 0)),
            pl.BlockSpec((g, d), lambda i: (0, 0)),
        ],
        out_shape=[
            jax.ShapeDtypeStruct((n, _NUM_CLASSES), jnp.float32),
            jax.ShapeDtypeStruct((32, n), jnp.float32),
            jax.ShapeDtypeStruct((g, d), jnp.float32),
        ],
        scratch_shapes=(
            [pltpu.VMEM((_BLK, d), jnp.float32) for _ in range(_NBUF)]
            + [pltpu.VMEM((n - (n // _BLK) * _BLK, d), jnp.float32)]
            + [pltpu.SemaphoreType.DMA for _ in range(_NBUF + 1)]
        ),
    )(pt, gt_bbox, ph, labs, x)

    overlap = cn
    masked_targets = mt.reshape(g, 4, n).transpose(0, 2, 1)
    return overlap, masked_targets, pf


# speedup vs baseline: 1.2140x; 1.2140x over previous
import jax
import jax.numpy as jnp
from jax.experimental import pallas as pl
from jax.experimental.pallas import tpu as pltpu

_NUM_CLASSES = 30
_LO = 1.0
_HI = 799.0
_BLK = 128
_SUB = 32
_NBUF = 39


def _body(pt_ref, gt_ref, ph_ref, lab_ref, x_ref, cn_ref, mt_ref, pf_ref,
          *rest):
    bufs = rest[:_NBUF]
    buft = rest[_NBUF]
    sems = rest[_NBUF + 1:2 * _NBUF + 1]
    semt = rest[2 * _NBUF + 1]
    n = pt_ref.shape[1]
    nfull = (n // _BLK) * _BLK

    px1 = jnp.clip(pt_ref[0:1, :], _LO, _HI)
    py1 = jnp.clip(pt_ref[1:2, :], _LO, _HI)
    px2 = jnp.clip(pt_ref[2:3, :], _LO, _HI)
    py2 = jnp.clip(pt_ref[3:4, :], _LO, _HI)
    area_b = (px2 - px1 + 1.0) * (py2 - py1 + 1.0)

    iou_rows = []
    for g in range(8):
        gx1 = jnp.clip(gt_ref[g, 0], _LO, _HI)
        gy1 = jnp.clip(gt_ref[g, 1], _LO, _HI)
        gx2 = jnp.clip(gt_ref[g, 2], _LO, _HI)
        gy2 = jnp.clip(gt_ref[g, 3], _LO, _HI)
        iw = jnp.maximum(jnp.minimum(px2, gx2) - jnp.maximum(px1, gx1)
                         + 1.0, 0.0)
        ih = jnp.maximum(jnp.minimum(py2, gy2) - jnp.maximum(py1, gy1)
                         + 1.0, 0.0)
        inter = iw * ih
        area_g = (gx2 - gx1 + 1.0) * (gy2 - gy1 + 1.0)
        iou_rows.append(inter / (area_b + area_g - inter))

    mrows = []
    for g in range(8):
        acc = iou_rows[g]
        for g2 in range(8):
            if g2 == g:
                continue
            same = lab_ref[g] == lab_ref[g2]
            acc = jnp.maximum(acc, jnp.where(same, iou_rows[g2], 0.0))
        mrows.append((acc > 0.6).astype(jnp.float32))
    mask = jnp.concatenate(mrows, axis=0)

    pf_ref[...] = jnp.zeros(pf_ref.shape, jnp.float32)

    nb = nfull // _BLK
    nsub = _BLK // _SUB
    subs = [[mask[:, b * _BLK + s * _SUB:b * _BLK + (s + 1) * _SUB]
             for s in range(nsub)] for b in range(nb)]
    sflags = [[jnp.max(subs[b][s]) > 0.5 for s in range(nsub)]
              for b in range(nb)]

    def start(b):
        for s in range(nsub):
            @pl.when(sflags[b][s])
            def _(s=s):
                pltpu.make_async_copy(
                    x_ref.at[pl.ds(b * _BLK + s * _SUB, _SUB), :],
                    bufs[b].at[pl.ds(s * _SUB, _SUB), :],
                    sems[b]).start()

    def finish(b):
        for s in range(nsub):
            @pl.when(sflags[b][s])
            def _(s=s):
                pltpu.make_async_copy(
                    x_ref.at[pl.ds(b * _BLK + s * _SUB, _SUB), :],
                    bufs[b].at[pl.ds(s * _SUB, _SUB), :],
                    sems[b]).wait()
                pf_ref[...] += jnp.dot(
                    subs[b][s], bufs[b][pl.ds(s * _SUB, _SUB), :],
                    preferred_element_type=jnp.float32)

    for b in range(nb):
        start(b)

    tail = mask[:, nfull:n] if nfull < n else None
    tflag2 = []
    if tail is not None:
        tflag2.append(jnp.max(tail) > 0.5)

        @pl.when(tflag2[0])
        def _():
            pltpu.make_async_copy(
                x_ref.at[pl.ds(nfull, n - nfull), :], buft, semt).start()

    cls_iota = jax.lax.broadcasted_iota(jnp.int32, (32, 1), 0)
    cn = jnp.zeros((32, n), jnp.float32)
    for g in range(8):
        onehot = (cls_iota == lab_ref[g]).astype(jnp.float32)
        cn = jnp.maximum(cn, onehot * iou_rows[g])
    cn_ref[...] = cn

    src_w = px2 - px1
    src_h = py2 - py1
    src_cx = px1 + 0.5 * src_w
    src_cy = py1 + 0.5 * src_h
    rows = []
    for g in range(8):
        hx1 = jnp.clip(ph_ref[g, 0], _LO, _HI)
        hy1 = jnp.clip(ph_ref[g, 1], _LO, _HI)
        hx2 = jnp.clip(ph_ref[g, 2], _LO, _HI)
        hy2 = jnp.clip(ph_ref[g, 3], _LO, _HI)
        gw = hx2 - hx1
        gh = hy2 - hy1
        gcx = hx1 + 0.5 * gw
        gcy = hy1 + 0.5 * gh
        m = mrows[g]
        rows.append(((gcx - src_cx) / src_w) * m)
        rows.append(((gcy - src_cy) / src_h) * m)
        rows.append(jnp.log(gw / src_w) * m)
        rows.append(jnp.log(gh / src_h) * m)
    mt_ref[...] = jnp.concatenate(rows, axis=0)

    for b in range(nb):
        finish(b)

    if tail is not None:
        @pl.when(tflag2[0])
        def _():
            pltpu.make_async_copy(
                x_ref.at[pl.ds(nfull, n - nfull), :], buft, semt).wait()
            pf_ref[...] += jnp.dot(tail, buft[...],
                                   preferred_element_type=jnp.float32)


def kernel(x, proposals, gt_bbox, gt_labels):
    n, d = x.shape
    g = gt_bbox.shape[0]
    labs = gt_labels.astype(jnp.int32)
    pt = proposals.T
    ph = proposals[:g]

    cn, mt, pf = pl.pallas_call(
        _body,
        grid=(1,),
        in_specs=[
            pl.BlockSpec((4, n), lambda i: (0, 0)),
            pl.BlockSpec(memory_space=pltpu.SMEM),
            pl.BlockSpec(memory_space=pltpu.SMEM),
            pl.BlockSpec(memory_space=pltpu.SMEM),
            pl.BlockSpec(memory_space=pltpu.MemorySpace.HBM),
        ],
        out_specs=[
            pl.BlockSpec((32, n), lambda i: (0, 0)),
            pl.BlockSpec((32, n), lambda i: (0, 0)),
            pl.BlockSpec((g, d), lambda i: (0, 0)),
        ],
        out_shape=[
            jax.ShapeDtypeStruct((32, n), jnp.float32),
            jax.ShapeDtypeStruct((32, n), jnp.float32),
            jax.ShapeDtypeStruct((g, d), jnp.float32),
        ],
        scratch_shapes=(
            [pltpu.VMEM((_BLK, d), jnp.float32) for _ in range(_NBUF)]
            + [pltpu.VMEM((n - (n // _BLK) * _BLK, d), jnp.float32)]
            + [pltpu.SemaphoreType.DMA for _ in range(_NBUF + 1)]
        ),
    )(pt, gt_bbox, ph, labs, x)

    overlap = cn[:_NUM_CLASSES].T
    masked_targets = mt.reshape(g, 4, n).transpose(0, 2, 1)
    return overlap, masked_targets, pf
